# baseline (speedup 1.0000x reference)
import functools

import jax
import jax.numpy as jnp
from jax import lax
from jax.experimental import pallas as pl
from jax.experimental.pallas import tpu as pltpu
from jax.experimental.pallas import tpu_sc as plsc

K_TOP = 200
NMS_THR = 0.5
B = 4
N = 5000
NP = 5120
KP = 208
L = 16
NC = 2
NW = 32
CHUNK = 64


def _nms_body(box_h, ord_h,
              keep_h, ret_h,
              vbox, vord,
              ky1, kx1, ky2, kx2, kar,
              okeep, oret,
              kcnt):
    c = lax.axis_index("c")
    s = lax.axis_index("s")
    wid = s * NC + c
    b = jnp.minimum(wid, B - 1)

    pltpu.sync_copy(box_h.at[b], vbox)
    pltpu.sync_copy(ord_h.at[b], vord)

    zf = jnp.zeros((L,), jnp.float32)
    zi = jnp.zeros((L,), jnp.int32)
    for t in range(KP // L):
        sl = pl.ds(t * L, L)
        ky1[sl] = zf
        kx1[sl] = zf
        ky2[sl] = zf
        kx2[sl] = zf
        kar[sl] = zf
        okeep[sl] = zi
        for cc in range(4):
            oret[cc, sl] = zf

    lanes = lax.iota(jnp.int32, L)
    lane0 = lanes == 0
    c0 = jnp.zeros((L,), jnp.int32)
    c1 = jnp.full((L,), 1, jnp.int32)
    c2 = jnp.full((L,), 2, jnp.int32)
    c3 = jnp.full((L,), 3, jnp.int32)

    kcnt[0] = jnp.int32(0)

    def pos_body(p, carry):
        kept = kcnt[0]
        pv = jnp.full((L,), p, jnp.int32)
        idxv = plsc.load_gather(vord, [pv])
        fb = idxv * 4
        y1c = plsc.load_gather(vbox, [fb])
        x1c = plsc.load_gather(vbox, [fb + c1])
        y2c = plsc.load_gather(vbox, [fb + c2])
        x2c = plsc.load_gather(vbox, [fb + c3])
        areac = (x2c - x1c) * (y2c - y1c)
        elig = jnp.logical_and(jnp.max(areac) >= 4.0, kept < K_TOP)

        nk = (kept + (L - 1)) // L

        def iou_step(t, miou):
            sl = pl.ds(t * L, L)
            a1 = ky1[sl]
            b1 = kx1[sl]
            a2 = ky2[sl]
            b2 = kx2[sl]
            ka = kar[sl]
            q_y1 = jnp.minimum(jnp.maximum(y1c, a1), a2)
            q_x1 = jnp.minimum(jnp.maximum(x1c, b1), b2)
            q_y2 = jnp.minimum(jnp.maximum(y2c, a1), a2)
            q_x2 = jnp.minimum(jnp.maximum(x2c, b1), b2)
            inter = (q_x2 - q_x1) * (q_y2 - q_y1)
            union = areac + ka - inter
            return jnp.maximum(miou, inter / union)

        miou = lax.fori_loop(0, nk, iou_step,
                             jnp.full((L,), -1.0, jnp.float32))
        take = jnp.logical_and(elig, jnp.max(miou) <= NMS_THR)

        @pl.when(take)
        def _accept():
            kv = jnp.full((L,), kept, jnp.int32)
            plsc.store_scatter(ky1, [kv], y1c, mask=lane0)
            plsc.store_scatter(kx1, [kv], x1c, mask=lane0)
            plsc.store_scatter(ky2, [kv], y2c, mask=lane0)
            plsc.store_scatter(kx2, [kv], x2c, mask=lane0)
            plsc.store_scatter(kar, [kv], areac, mask=lane0)
            plsc.store_scatter(okeep, [kv], idxv, mask=lane0)
            plsc.store_scatter(oret, [c0, kv], y1c, mask=lane0)
            plsc.store_scatter(oret, [c1, kv], x1c, mask=lane0)
            plsc.store_scatter(oret, [c2, kv], y2c, mask=lane0)
            plsc.store_scatter(oret, [c3, kv], x2c, mask=lane0)
            kcnt[0] = kept + 1

        return carry

    def blk_body(t, carry):
        @pl.when(kcnt[0] < K_TOP)
        def _blk():
            lax.fori_loop(t * CHUNK, (t + 1) * CHUNK, pos_body,
                          jnp.int32(0))
        return carry

    lax.fori_loop(0, NP // CHUNK, blk_body, jnp.int32(0))

    plsc.store_scatter(okeep, [jnp.full((L,), K_TOP, jnp.int32)],
                       jnp.full((L,), kcnt[0], jnp.int32), mask=lane0)

    pltpu.sync_copy(okeep, keep_h.at[wid])
    pltpu.sync_copy(oret, ret_h.at[wid])


_nms_sc = functools.partial(
    pl.kernel,
    out_type=(
        jax.ShapeDtypeStruct((NW, KP), jnp.int32),
        jax.ShapeDtypeStruct((NW, 4, KP), jnp.float32),
    ),
    mesh=plsc.VectorSubcoreMesh(core_axis_name="c", subcore_axis_name="s"),
    scratch_types=[
        pltpu.VMEM((NP * 4,), jnp.float32),
        pltpu.VMEM((NP,), jnp.int32),
        pltpu.VMEM((KP,), jnp.float32),
        pltpu.VMEM((KP,), jnp.float32),
        pltpu.VMEM((KP,), jnp.float32),
        pltpu.VMEM((KP,), jnp.float32),
        pltpu.VMEM((KP,), jnp.float32),
        pltpu.VMEM((KP,), jnp.int32),
        pltpu.VMEM((4, KP), jnp.float32),
        pltpu.SMEM((1,), jnp.int32),
    ],
    compiler_params=pltpu.CompilerParams(needs_layout_passes=False),
)(_nms_body)


def kernel(scoress, bboxess):
    probs = jax.nn.softmax(scoress, axis=2)
    sc = probs[:, :, 0]
    order_desc = jnp.argsort(sc, axis=1, stable=True)[:, ::-1].astype(jnp.int32)

    orderp = jnp.pad(order_desc, ((0, 0), (0, NP - N)), constant_values=N)
    boxp = jnp.pad(bboxess, ((0, 0), (0, NP - N), (0, 0))).reshape(B, NP * 4)

    okeep, oret = _nms_sc(boxp, orderp)

    keeps = okeep[:B, :K_TOP].astype(jnp.int64)
    counts = okeep[:B, K_TOP:K_TOP + 1].astype(jnp.int64)
    ret = jnp.transpose(oret[:B, :, :K_TOP], (0, 2, 1))
    return (ret, counts, keeps)


# speedup vs baseline: 1.3279x; 1.3279x over previous
import functools

import jax
import jax.numpy as jnp
from jax import lax
from jax.experimental import pallas as pl
from jax.experimental.pallas import tpu as pltpu
from jax.experimental.pallas import tpu_sc as plsc

K_TOP = 200
NMS_THR = 0.5
B = 4
N = 5000
NP = 5120
KP = 208
L = 16
NC = 2
NW = 32
CHUNK = 64


def _nms_body(y1_h, x1_h, y2_h, x2_h, ord_h,
              keep_h, ry1_h, rx1_h, ry2_h, rx2_h, cnt_h,
              vy1, vx1, vy2, vx2, vord,
              ky1, kx1, ky2, kx2, kar,
              okeep, oy1, ox1, oy2, ox2, ocnt):
    c = lax.axis_index("c")
    s = lax.axis_index("s")
    wid = s * NC + c
    b = jnp.minimum(wid, B - 1)

    pltpu.sync_copy(y1_h.at[b], vy1)
    pltpu.sync_copy(x1_h.at[b], vx1)
    pltpu.sync_copy(y2_h.at[b], vy2)
    pltpu.sync_copy(x2_h.at[b], vx2)
    pltpu.sync_copy(ord_h.at[b], vord)

    zf = jnp.zeros((L,), jnp.float32)
    zi = jnp.zeros((L,), jnp.int32)
    for t in range(KP // L):
        sl = pl.ds(t * L, L)
        ky1[sl] = zf
        kx1[sl] = zf
        ky2[sl] = zf
        kx2[sl] = zf
        kar[sl] = zf
        okeep[sl] = zi
        oy1[sl] = zf
        ox1[sl] = zf
        oy2[sl] = zf
        ox2[sl] = zf

    lanes = lax.iota(jnp.int32, L)
    lane0 = lanes == 0

    def pos_body(p, kept):
        pv = jnp.full((L,), p, jnp.int32)
        idxv = plsc.load_gather(vord, [pv])
        y1c = plsc.load_gather(vy1, [idxv])
        x1c = plsc.load_gather(vx1, [idxv])
        y2c = plsc.load_gather(vy2, [idxv])
        x2c = plsc.load_gather(vx2, [idxv])
        areac = (x2c - x1c) * (y2c - y1c)
        miou0 = jnp.where(areac >= 4.0, jnp.full((L,), -1.0, jnp.float32),
                          jnp.full((L,), 2.0, jnp.float32))

        nk = jnp.where(kept < K_TOP, (kept + (L - 1)) // L, 0)

        def iou_step(t, miou):
            sl = pl.ds(t * L, L)
            a1 = ky1[sl]
            b1 = kx1[sl]
            a2 = ky2[sl]
            b2 = kx2[sl]
            ka = kar[sl]
            q_y1 = jnp.minimum(jnp.maximum(y1c, a1), a2)
            q_x1 = jnp.minimum(jnp.maximum(x1c, b1), b2)
            q_y2 = jnp.minimum(jnp.maximum(y2c, a1), a2)
            q_x2 = jnp.minimum(jnp.maximum(x2c, b1), b2)
            inter = (q_x2 - q_x1) * (q_y2 - q_y1)
            union = areac + ka - inter
            return jnp.maximum(miou, inter / union)

        miou = lax.fori_loop(0, nk, iou_step, miou0)
        take = jnp.logical_and(kept < K_TOP, jnp.max(miou) <= NMS_THR)

        @pl.when(take)
        def _accept():
            kv = jnp.full((L,), kept, jnp.int32)
            plsc.store_scatter(ky1, [kv], y1c, mask=lane0)
            plsc.store_scatter(kx1, [kv], x1c, mask=lane0)
            plsc.store_scatter(ky2, [kv], y2c, mask=lane0)
            plsc.store_scatter(kx2, [kv], x2c, mask=lane0)
            plsc.store_scatter(kar, [kv], areac, mask=lane0)
            plsc.store_scatter(okeep, [kv], idxv, mask=lane0)
            plsc.store_scatter(oy1, [kv], y1c, mask=lane0)
            plsc.store_scatter(ox1, [kv], x1c, mask=lane0)
            plsc.store_scatter(oy2, [kv], y2c, mask=lane0)
            plsc.store_scatter(ox2, [kv], x2c, mask=lane0)

        return jnp.where(take, kept + 1, kept)

    def chunk_body(t, kept):
        return lax.cond(kept < K_TOP,
                        lambda k: lax.fori_loop(t * CHUNK, (t + 1) * CHUNK,
                                                pos_body, k),
                        lambda k: k,
                        kept)

    kept = lax.fori_loop(0, NP // CHUNK, chunk_body, jnp.int32(0))

    ocnt[...] = jnp.full((L,), kept, jnp.int32)

    pltpu.sync_copy(okeep, keep_h.at[wid])
    pltpu.sync_copy(oy1, ry1_h.at[wid])
    pltpu.sync_copy(ox1, rx1_h.at[wid])
    pltpu.sync_copy(oy2, ry2_h.at[wid])
    pltpu.sync_copy(ox2, rx2_h.at[wid])
    pltpu.sync_copy(ocnt, cnt_h.at[wid])


_nms_sc = functools.partial(
    pl.kernel,
    out_type=(
        jax.ShapeDtypeStruct((NW, KP), jnp.int32),
        jax.ShapeDtypeStruct((NW, KP), jnp.float32),
        jax.ShapeDtypeStruct((NW, KP), jnp.float32),
        jax.ShapeDtypeStruct((NW, KP), jnp.float32),
        jax.ShapeDtypeStruct((NW, KP), jnp.float32),
        jax.ShapeDtypeStruct((NW, L), jnp.int32),
    ),
    mesh=plsc.VectorSubcoreMesh(core_axis_name="c", subcore_axis_name="s"),
    scratch_types=[
        pltpu.VMEM((NP,), jnp.float32),
        pltpu.VMEM((NP,), jnp.float32),
        pltpu.VMEM((NP,), jnp.float32),
        pltpu.VMEM((NP,), jnp.float32),
        pltpu.VMEM((NP,), jnp.int32),
        pltpu.VMEM((KP,), jnp.float32),
        pltpu.VMEM((KP,), jnp.float32),
        pltpu.VMEM((KP,), jnp.float32),
        pltpu.VMEM((KP,), jnp.float32),
        pltpu.VMEM((KP,), jnp.float32),
        pltpu.VMEM((KP,), jnp.int32),
        pltpu.VMEM((KP,), jnp.float32),
        pltpu.VMEM((KP,), jnp.float32),
        pltpu.VMEM((KP,), jnp.float32),
        pltpu.VMEM((KP,), jnp.float32),
        pltpu.VMEM((L,), jnp.int32),
    ],
    compiler_params=pltpu.CompilerParams(needs_layout_passes=False),
)(_nms_body)


def kernel(scoress, bboxess):
    probs = jax.nn.softmax(scoress, axis=2)
    sc = probs[:, :, 0]
    order_desc = jnp.argsort(sc, axis=1, stable=True)[:, ::-1].astype(jnp.int32)

    pad = ((0, 0), (0, NP - N))
    y1 = jnp.pad(bboxess[:, :, 0], pad)
    x1 = jnp.pad(bboxess[:, :, 1], pad)
    y2 = jnp.pad(bboxess[:, :, 2], pad)
    x2 = jnp.pad(bboxess[:, :, 3], pad)
    orderp = jnp.pad(order_desc, pad, constant_values=N)

    okeep, oy1, ox1, oy2, ox2, ocnt = _nms_sc(y1, x1, y2, x2, orderp)

    keeps = okeep[:B, :K_TOP].astype(jnp.int64)
    counts = ocnt[:B, :1].astype(jnp.int64)
    ret = jnp.stack([oy1[:B, :K_TOP], ox1[:B, :K_TOP],
                     oy2[:B, :K_TOP], ox2[:B, :K_TOP]], axis=-1)
    return (ret, counts, keeps)


# speedup vs baseline: 1.3387x; 1.0081x over previous
import functools

import jax
import jax.numpy as jnp
from jax import lax
from jax.experimental import pallas as pl
from jax.experimental.pallas import tpu as pltpu
from jax.experimental.pallas import tpu_sc as plsc

K_TOP = 200
NMS_THR = 0.5
B = 4
N = 5000
NP = 5120
KP = 208
L = 16
NC = 2
NW = 32
CHUNK = 64


def _nms_body(y1_h, x1_h, y2_h, x2_h, ord_h,
              keep_h, ry1_h, rx1_h, ry2_h, rx2_h, cnt_h,
              vy1, vx1, vy2, vx2, vord,
              ky1, kx1, ky2, kx2, kar,
              okeep, oy1, ox1, oy2, ox2, ocnt):
    c = lax.axis_index("c")
    s = lax.axis_index("s")
    wid = s * NC + c
    b = jnp.minimum(wid, B - 1)

    pltpu.sync_copy(y1_h.at[b], vy1)
    pltpu.sync_copy(x1_h.at[b], vx1)
    pltpu.sync_copy(y2_h.at[b], vy2)
    pltpu.sync_copy(x2_h.at[b], vx2)
    pltpu.sync_copy(ord_h.at[b], vord)

    zf = jnp.zeros((L,), jnp.float32)
    zi = jnp.zeros((L,), jnp.int32)
    for t in range(KP // L):
        sl = pl.ds(t * L, L)
        ky1[sl] = zf
        kx1[sl] = zf
        ky2[sl] = zf
        kx2[sl] = zf
        kar[sl] = zf
        okeep[sl] = zi
        oy1[sl] = zf
        ox1[sl] = zf
        oy2[sl] = zf
        ox2[sl] = zf

    lanes = lax.iota(jnp.int32, L)
    lane0 = lanes == 0

    def load_cand(p):
        pv = jnp.full((L,), p, jnp.int32)
        idxv = plsc.load_gather(vord, [pv])
        y1c = plsc.load_gather(vy1, [idxv])
        x1c = plsc.load_gather(vx1, [idxv])
        y2c = plsc.load_gather(vy2, [idxv])
        x2c = plsc.load_gather(vx2, [idxv])
        return (idxv, y1c, x1c, y2c, x2c)

    def pos_body(p, state):
        kept, cur = state
        idxv, y1c, x1c, y2c, x2c = cur
        nxt = load_cand(jnp.minimum(p + 1, NP - 1))
        areac = (x2c - x1c) * (y2c - y1c)
        miou0 = jnp.where(areac >= 4.0, jnp.full((L,), -1.0, jnp.float32),
                          jnp.full((L,), 2.0, jnp.float32))

        nk = jnp.where(kept < K_TOP, (kept + (L - 1)) // L, 0)

        def iou_step(t, miou):
            sl = pl.ds(t * L, L)
            a1 = ky1[sl]
            b1 = kx1[sl]
            a2 = ky2[sl]
            b2 = kx2[sl]
            ka = kar[sl]
            q_y1 = jnp.minimum(jnp.maximum(y1c, a1), a2)
            q_x1 = jnp.minimum(jnp.maximum(x1c, b1), b2)
            q_y2 = jnp.minimum(jnp.maximum(y2c, a1), a2)
            q_x2 = jnp.minimum(jnp.maximum(x2c, b1), b2)
            inter = (q_x2 - q_x1) * (q_y2 - q_y1)
            union = areac + ka - inter
            return jnp.maximum(miou, inter / union)

        miou = lax.fori_loop(0, nk, iou_step, miou0)
        take = jnp.logical_and(kept < K_TOP, jnp.max(miou) <= NMS_THR)

        @pl.when(take)
        def _accept():
            kv = jnp.full((L,), kept, jnp.int32)
            plsc.store_scatter(ky1, [kv], y1c, mask=lane0)
            plsc.store_scatter(kx1, [kv], x1c, mask=lane0)
            plsc.store_scatter(ky2, [kv], y2c, mask=lane0)
            plsc.store_scatter(kx2, [kv], x2c, mask=lane0)
            plsc.store_scatter(kar, [kv], areac, mask=lane0)
            plsc.store_scatter(okeep, [kv], idxv, mask=lane0)
            plsc.store_scatter(oy1, [kv], y1c, mask=lane0)
            plsc.store_scatter(ox1, [kv], x1c, mask=lane0)
            plsc.store_scatter(oy2, [kv], y2c, mask=lane0)
            plsc.store_scatter(ox2, [kv], x2c, mask=lane0)

        return (jnp.where(take, kept + 1, kept), nxt)

    def chunk_body(t, state):
        return lax.cond(state[0] < K_TOP,
                        lambda st: lax.fori_loop(t * CHUNK, (t + 1) * CHUNK,
                                                 pos_body, st),
                        lambda st: st,
                        state)

    kept, _ = lax.fori_loop(0, NP // CHUNK, chunk_body,
                            (jnp.int32(0), load_cand(0)))

    ocnt[...] = jnp.full((L,), kept, jnp.int32)

    pltpu.sync_copy(okeep, keep_h.at[wid])
    pltpu.sync_copy(oy1, ry1_h.at[wid])
    pltpu.sync_copy(ox1, rx1_h.at[wid])
    pltpu.sync_copy(oy2, ry2_h.at[wid])
    pltpu.sync_copy(ox2, rx2_h.at[wid])
    pltpu.sync_copy(ocnt, cnt_h.at[wid])


_nms_sc = functools.partial(
    pl.kernel,
    out_type=(
        jax.ShapeDtypeStruct((NW, KP), jnp.int32),
        jax.ShapeDtypeStruct((NW, KP), jnp.float32),
        jax.ShapeDtypeStruct((NW, KP), jnp.float32),
        jax.ShapeDtypeStruct((NW, KP), jnp.float32),
        jax.ShapeDtypeStruct((NW, KP), jnp.float32),
        jax.ShapeDtypeStruct((NW, L), jnp.int32),
    ),
    mesh=plsc.VectorSubcoreMesh(core_axis_name="c", subcore_axis_name="s"),
    scratch_types=[
        pltpu.VMEM((NP,), jnp.float32),
        pltpu.VMEM((NP,), jnp.float32),
        pltpu.VMEM((NP,), jnp.float32),
        pltpu.VMEM((NP,), jnp.float32),
        pltpu.VMEM((NP,), jnp.int32),
        pltpu.VMEM((KP,), jnp.float32),
        pltpu.VMEM((KP,), jnp.float32),
        pltpu.VMEM((KP,), jnp.float32),
        pltpu.VMEM((KP,), jnp.float32),
        pltpu.VMEM((KP,), jnp.float32),
        pltpu.VMEM((KP,), jnp.int32),
        pltpu.VMEM((KP,), jnp.float32),
        pltpu.VMEM((KP,), jnp.float32),
        pltpu.VMEM((KP,), jnp.float32),
        pltpu.VMEM((KP,), jnp.float32),
        pltpu.VMEM((L,), jnp.int32),
    ],
    compiler_params=pltpu.CompilerParams(needs_layout_passes=False),
)(_nms_body)


def kernel(scoress, bboxess):
    probs = jax.nn.softmax(scoress, axis=2)
    sc = probs[:, :, 0]
    order_desc = jnp.argsort(sc, axis=1, stable=True)[:, ::-1].astype(jnp.int32)

    pad = ((0, 0), (0, NP - N))
    y1 = jnp.pad(bboxess[:, :, 0], pad)
    x1 = jnp.pad(bboxess[:, :, 1], pad)
    y2 = jnp.pad(bboxess[:, :, 2], pad)
    x2 = jnp.pad(bboxess[:, :, 3], pad)
    orderp = jnp.pad(order_desc, pad, constant_values=N)

    okeep, oy1, ox1, oy2, ox2, ocnt = _nms_sc(y1, x1, y2, x2, orderp)

    keeps = okeep[:B, :K_TOP].astype(jnp.int64)
    counts = ocnt[:B, :1].astype(jnp.int64)
    ret = jnp.stack([oy1[:B, :K_TOP], ox1[:B, :K_TOP],
                     oy2[:B, :K_TOP], ox2[:B, :K_TOP]], axis=-1)
    return (ret, counts, keeps)
